# Initial kernel scaffold; baseline (speedup 1.0000x reference)
import jax
import jax.numpy as jnp
from jax.experimental import pallas as pl


def kernel(z, edge_index):
    raise NotImplementedError("write your pallas kernel here")


# speedup vs baseline: 2.0400x; 2.0400x over previous
import functools

import jax
import jax.numpy as jnp
from jax import lax
from jax.experimental import pallas as pl
from jax.experimental.pallas import tpu as pltpu
from jax.experimental.pallas import tpu_sc as plsc

N_NODES = 10000
D_FEAT = 128
N_EDGES = 320000

NC, NS, L = 2, 16, 16
NW = NC * NS
EW = N_EDGES // NW
B = 80
NB = EW // B
G = B // L
C = D_FEAT // L

_mesh = plsc.VectorSubcoreMesh(core_axis_name="c", subcore_axis_name="s")

_DNUMS = lax.GatherDimensionNumbers(
    offset_dims=(), collapsed_slice_dims=(0,), start_index_map=(0,))


def _take16(x, idx2d):
    return lax.gather(x, idx2d, _DNUMS, slice_sizes=(1,),
                      mode=lax.GatherScatterMode.PROMISE_IN_BOUNDS)


@functools.partial(
    pl.kernel,
    out_type=jax.ShapeDtypeStruct((N_EDGES,), jnp.float32),
    mesh=_mesh,
    scratch_types=[
        pltpu.VMEM((B,), jnp.int32),
        pltpu.VMEM((B,), jnp.int32),
        pltpu.VMEM((B, D_FEAT), jnp.float32),
        pltpu.VMEM((B, D_FEAT), jnp.float32),
        pltpu.VMEM((B,), jnp.float32),
        pltpu.SemaphoreType.DMA,
        pltpu.SemaphoreType.DMA,
    ],
)
def _gae_decode(z_hbm, ei_hbm, out_hbm, idx_s, idx_d, rows_s, rows_d,
                out_v, sem_s, sem_d):
    wid = lax.axis_index("s") * NC + lax.axis_index("c")
    base = wid * EW
    lane = lax.iota(jnp.int32, L)
    perms = [(lax.iota(jnp.int32, L) ^ sh)[:, None] for sh in (8, 4, 2, 1)]

    def body(g, _):
        start = base + g * B
        pltpu.sync_copy(ei_hbm.at[pl.ds(start, B)], idx_s)
        pltpu.sync_copy(ei_hbm.at[pl.ds(N_EDGES + start, B)], idx_d)
        cs = pltpu.async_copy(z_hbm.at[idx_s], rows_s, sem_s)
        cd = pltpu.async_copy(z_hbm.at[idx_d], rows_d, sem_d)
        cs.wait()
        cd.wait()
        for grp in range(G):
            out_vec = jnp.zeros((L,), jnp.float32)
            for e in range(L):
                row = grp * L + e
                acc = rows_s[row, pl.ds(0, L)] * rows_d[row, pl.ds(0, L)]
                for c in range(1, C):
                    acc += (rows_s[row, pl.ds(c * L, L)]
                            * rows_d[row, pl.ds(c * L, L)])
                for p in perms:
                    acc = acc + _take16(acc, p)
                out_vec = jnp.where(lane == e, acc, out_vec)
            out_v[pl.ds(grp * L, L)] = 1.0 / (1.0 + jnp.exp(-out_vec))
        pltpu.sync_copy(out_v, out_hbm.at[pl.ds(start, B)])
        return 0

    lax.fori_loop(0, NB, body, 0)


def kernel(z, edge_index):
    return _gae_decode(z, edge_index.astype(jnp.int32).reshape(-1))


# speedup vs baseline: 3.7654x; 1.8457x over previous
import functools

import jax
import jax.numpy as jnp
from jax import lax
from jax.experimental import pallas as pl
from jax.experimental.pallas import tpu as pltpu
from jax.experimental.pallas import tpu_sc as plsc

N_NODES = 10000
D_FEAT = 128
N_EDGES = 320000

NC, NS, L = 2, 16, 16
NW = NC * NS
EW = N_EDGES // NW
B = 80
NB = EW // B
G = B // L
C = D_FEAT // L

_mesh = plsc.VectorSubcoreMesh(core_axis_name="c", subcore_axis_name="s")

_DNUMS = lax.GatherDimensionNumbers(
    offset_dims=(), collapsed_slice_dims=(0,), start_index_map=(0,))


def _take16(x, idx2d):
    return lax.gather(x, idx2d, _DNUMS, slice_sizes=(1,),
                      mode=lax.GatherScatterMode.PROMISE_IN_BOUNDS)


@functools.partial(
    pl.kernel,
    out_type=jax.ShapeDtypeStruct((N_EDGES,), jnp.float32),
    mesh=_mesh,
    scratch_types=[
        pltpu.VMEM((EW,), jnp.int32),
        pltpu.VMEM((EW,), jnp.int32),
        pltpu.VMEM((B, D_FEAT), jnp.float32),
        pltpu.VMEM((B, D_FEAT), jnp.float32),
        pltpu.VMEM((B, D_FEAT), jnp.float32),
        pltpu.VMEM((B, D_FEAT), jnp.float32),
        pltpu.VMEM((EW,), jnp.float32),
        pltpu.SemaphoreType.DMA,
        pltpu.SemaphoreType.DMA,
        pltpu.SemaphoreType.DMA,
        pltpu.SemaphoreType.DMA,
    ],
)
def _gae_decode(z_hbm, ei_hbm, out_hbm, idx_s, idx_d, rs0, rs1, rd0, rd1,
                out_v, ss0, ss1, sd0, sd1):
    wid = lax.axis_index("s") * NC + lax.axis_index("c")
    base = wid * EW
    lane = lax.iota(jnp.int32, L)
    perms = [(lax.iota(jnp.int32, L) ^ sh)[:, None] for sh in (8, 4, 2, 1)]
    bufs = ((rs0, rd0, ss0, sd0), (rs1, rd1, ss1, sd1))

    pltpu.sync_copy(ei_hbm.at[pl.ds(base, EW)], idx_s)
    pltpu.sync_copy(ei_hbm.at[pl.ds(N_EDGES + base, EW)], idx_d)

    def fire(g, slot):
        rs, rd, ss, sd = bufs[slot]
        pltpu.async_copy(z_hbm.at[idx_s.at[pl.ds(g * B, B)]], rs, ss)
        pltpu.async_copy(z_hbm.at[idx_d.at[pl.ds(g * B, B)]], rd, sd)

    def consume(g, slot):
        rs, rd, ss, sd = bufs[slot]
        pltpu.make_async_copy(z_hbm.at[idx_s.at[pl.ds(0, B)]], rs, ss).wait()
        pltpu.make_async_copy(z_hbm.at[idx_d.at[pl.ds(0, B)]], rd, sd).wait()
        for grp in range(G):
            out_vec = jnp.zeros((L,), jnp.float32)
            for e in range(L):
                row = grp * L + e
                acc = rs[row, pl.ds(0, L)] * rd[row, pl.ds(0, L)]
                for c in range(1, C):
                    acc += rs[row, pl.ds(c * L, L)] * rd[row, pl.ds(c * L, L)]
                for p in perms:
                    acc = acc + _take16(acc, p)
                out_vec = jnp.where(lane == e, acc, out_vec)
            sig = 1.0 / (1.0 + jnp.exp(-out_vec))
            out_v[pl.ds(g * B + grp * L, L)] = sig

    fire(0, 0)
    fire(1, 1)

    def body(k, _):
        g = k * 2
        consume(g, 0)

        @pl.when(g + 2 < NB)
        def _():
            fire(g + 2, 0)

        consume(g + 1, 1)

        @pl.when(g + 3 < NB)
        def _():
            fire(g + 3, 1)

        return 0

    lax.fori_loop(0, NB // 2, body, 0)
    consume(NB - 1, 0)

    pltpu.sync_copy(out_v, out_hbm.at[pl.ds(base, EW)])


def kernel(z, edge_index):
    return _gae_decode(z, edge_index.astype(jnp.int32).reshape(-1))
